# baseline (speedup 1.0000x reference)
import functools

import jax
import jax.numpy as jnp
from jax import lax
from jax.experimental import pallas as pl
from jax.experimental.pallas import tpu as pltpu
from jax.experimental.pallas import tpu_sc as plsc

D = 768
FF = 3072
E = 8
K = 2
T = 2048

TILE_M = 512
TSH = 9
P = T * K + E * TILE_M
NT = P // TILE_M

_BT1 = 256
_BT4 = 256

NW = 32
SLOTS = P // NW
GCH = 64
PAIRS = (T * K) // NW


def _k1_body(x_ref, wg_ref, g_ref, b_ref, h_ref, ti_ref, tw_ref):
    x = x_ref[...]
    mu = jnp.mean(x, axis=-1, keepdims=True)
    var = jnp.mean((x - mu) ** 2, axis=-1, keepdims=True)
    h = (x - mu) / jnp.sqrt(var + 1e-5) * g_ref[...] + b_ref[...]
    h_ref[...] = h
    logits = jnp.dot(h, wg_ref[...], preferred_element_type=jnp.float32)
    m = jnp.max(logits, axis=-1, keepdims=True)
    ex = jnp.exp(logits - m)
    p = ex / jnp.sum(ex, axis=-1, keepdims=True)
    ei = jax.lax.broadcasted_iota(jnp.int32, (_BT1, E), 1)
    v1 = jnp.max(p, axis=-1, keepdims=True)
    i1 = jnp.min(jnp.where(p == v1, ei, E), axis=-1, keepdims=True)
    p2 = jnp.where(ei == i1, -1.0, p)
    v2 = jnp.max(p2, axis=-1, keepdims=True)
    i2 = jnp.min(jnp.where(p2 == v2, ei, E), axis=-1, keepdims=True)
    den = v1 + v2 + 1e-9
    ti_ref[...] = jnp.concatenate([i1, i2], axis=1)
    tw_ref[...] = jnp.concatenate([v1 / den, v2 / den], axis=1)


def _route_body(ef_ref, pos_ref, meta_ref):
    e = ef_ref[...]
    n = T * K
    se = jax.lax.broadcasted_iota(jnp.int32, (E, n), 0)
    oh = (e == se).astype(jnp.int32)
    c = oh
    s = 1
    while s < n:
        z = jnp.zeros((E, s), jnp.int32)
        c = c + jnp.concatenate([z, c[:, : n - s]], axis=1)
        s *= 2
    rank = jnp.sum(oh * c, axis=0, keepdims=True) - 1
    counts = c[:, n - 1 : n]
    psz = ((counts + (TILE_M - 1)) >> TSH) << TSH
    q = psz
    s = 1
    while s < E:
        z = jnp.zeros((s, 1), jnp.int32)
        q = q + jnp.concatenate([z, q[: E - s, :]], axis=0)
        s *= 2
    off = q - psz
    pos_ref[...] = jnp.sum(oh * off, axis=0, keepdims=True) + rank
    p_used = q[E - 1 : E, :]
    ti = jax.lax.broadcasted_iota(jnp.int32, (1, 64), 1) * TILE_M
    texp = jnp.zeros((1, 64), jnp.int32)
    for ee in range(1, E):
        texp = texp + (ti >= off[ee : ee + 1, :]).astype(jnp.int32)
    nv = p_used >> TSH
    li = jax.lax.broadcasted_iota(jnp.int32, (1, 64), 1)
    meta_ref[...] = jnp.where(li == 63, nv, texp)


def _sca_body(pos_hbm, w_hbm, h_hbm, hs_hbm, ws_hbm,
              pos_v, w_v, ri_v, wl_v, rows_v, sem):
    wid = lax.axis_index("s") * 2 + lax.axis_index("c")
    base = wid * SLOTS
    pltpu.sync_copy(pos_hbm, pos_v)
    pltpu.sync_copy(w_hbm, w_v)
    zi = jnp.zeros((16,), jnp.int32)
    zf = jnp.zeros((16,), jnp.float32)
    for j in range(SLOTS // 16):
        ri_v[pl.ds(j * 16, 16)] = zi
        wl_v[pl.ds(j * 16, 16)] = zf
    lane = lax.broadcasted_iota(jnp.int32, (16,), 0)

    def scan(c, pmax):
        pp = pos_v[pl.ds(c * 16, 16)]
        idx = pp - base
        msk = (idx >= 0) & (idx < SLOTS)
        idx = jnp.clip(idx, 0, SLOTS - 1)
        tok = (c * 16 + lane) >> 1
        plsc.store_scatter(ri_v, [idx], tok, mask=msk)
        plsc.store_scatter(wl_v, [idx], w_v[pl.ds(c * 16, 16)], mask=msk)
        return jnp.maximum(pmax, jnp.max(pp))

    pmax = lax.fori_loop(0, (T * K) // 16, scan, jnp.int32(0))

    pltpu.sync_copy(wl_v, ws_hbm.at[pl.ds(base, SLOTS)])
    for g in range(SLOTS // GCH):
        @pl.when(base + g * GCH <= pmax)
        def _chunk():
            idx_ref = ri_v.at[pl.ds(g * GCH, GCH)]
            pltpu.async_copy(h_hbm.at[idx_ref], rows_v, sem).wait()
            pltpu.sync_copy(rows_v, hs_hbm.at[pl.ds(base + g * GCH, GCH)])


def _scb_body(pos_hbm, ys_hbm, ysg_hbm, idx_v, rows_v, sem):
    wid = lax.axis_index("s") * 2 + lax.axis_index("c")
    base = wid * PAIRS
    pltpu.sync_copy(pos_hbm.at[pl.ds(base, PAIRS)], idx_v)
    pltpu.async_copy(ys_hbm.at[idx_v], rows_v, sem).wait()
    pltpu.sync_copy(rows_v, ysg_hbm.at[pl.ds(base, PAIRS)])


def _k3_body(meta_ref, hs_ref, ws_ref, w1_ref, b1_ref, w2_ref, b2_ref,
             ys_ref):
    i = pl.program_id(0)
    nv = meta_ref[63]

    @pl.when(i < nv)
    def _ffn():
        hs_bf = hs_ref[...].astype(jnp.bfloat16)
        a = jnp.maximum(
            jnp.dot(hs_bf, w1_ref[0], preferred_element_type=jnp.float32)
            + b1_ref[0], 0.0)
        y = jnp.dot(a.astype(jnp.bfloat16), w2_ref[0],
                    preferred_element_type=jnp.float32) + b2_ref[0]
        ys_ref[...] = y * ws_ref[...]


def _k4_body(x_ref, y_ref, o_ref):
    o_ref[...] = x_ref[...] + y_ref[:, :D] + y_ref[:, D:]


def kernel(x, Wg, W1, b1, W2, b2, gamma, beta):
    g2 = gamma.reshape(1, D)
    bt2 = beta.reshape(1, D)

    h, tidx, tw = pl.pallas_call(
        _k1_body,
        grid=(T // _BT1,),
        in_specs=[
            pl.BlockSpec((_BT1, D), lambda i: (i, 0)),
            pl.BlockSpec((D, E), lambda i: (0, 0)),
            pl.BlockSpec((1, D), lambda i: (0, 0)),
            pl.BlockSpec((1, D), lambda i: (0, 0)),
        ],
        out_specs=[
            pl.BlockSpec((_BT1, D), lambda i: (i, 0)),
            pl.BlockSpec((_BT1, K), lambda i: (i, 0)),
            pl.BlockSpec((_BT1, K), lambda i: (i, 0)),
        ],
        out_shape=[
            jax.ShapeDtypeStruct((T, D), jnp.float32),
            jax.ShapeDtypeStruct((T, K), jnp.int32),
            jax.ShapeDtypeStruct((T, K), jnp.float32),
        ],
    )(x, Wg, g2, bt2)

    ef = tidx.reshape(1, T * K)
    pos1, meta = pl.pallas_call(
        _route_body,
        in_specs=[pl.BlockSpec((1, T * K), lambda: (0, 0))],
        out_specs=[
            pl.BlockSpec((1, T * K), lambda: (0, 0)),
            pl.BlockSpec((1, 64), lambda: (0, 0)),
        ],
        out_shape=[
            jax.ShapeDtypeStruct((1, T * K), jnp.int32),
            jax.ShapeDtypeStruct((1, 64), jnp.int32),
        ],
    )(ef)

    posf = pos1.reshape(T * K)
    wf = tw.reshape(T * K)
    meta1 = meta.reshape(64)

    sca = pl.kernel(
        _sca_body,
        out_type=[
            jax.ShapeDtypeStruct((P, D), jnp.float32),
            jax.ShapeDtypeStruct((P,), jnp.float32),
        ],
        mesh=plsc.VectorSubcoreMesh(core_axis_name="c", subcore_axis_name="s"),
        compiler_params=pltpu.CompilerParams(needs_layout_passes=False),
        scratch_types=[
            pltpu.VMEM((T * K,), jnp.int32),
            pltpu.VMEM((T * K,), jnp.float32),
            pltpu.VMEM((SLOTS,), jnp.int32),
            pltpu.VMEM((SLOTS,), jnp.float32),
            pltpu.VMEM((GCH, D), jnp.float32),
            pltpu.SemaphoreType.DMA,
        ],
    )
    hs, ws = sca(posf, wf, h)

    w1b = W1.astype(jnp.bfloat16)
    w2b = W2.astype(jnp.bfloat16)

    ys = pl.pallas_call(
        _k3_body,
        grid_spec=pltpu.PrefetchScalarGridSpec(
            num_scalar_prefetch=1,
            grid=(NT,),
            in_specs=[
                pl.BlockSpec((TILE_M, D), lambda i, m: (i, 0)),
                pl.BlockSpec((TILE_M, 1), lambda i, m: (i, 0)),
                pl.BlockSpec((1, D, FF), lambda i, m: (m[i], 0, 0)),
                pl.BlockSpec((1, 1, FF), lambda i, m: (m[i], 0, 0)),
                pl.BlockSpec((1, FF, D), lambda i, m: (m[i], 0, 0)),
                pl.BlockSpec((1, 1, D), lambda i, m: (m[i], 0, 0)),
            ],
            out_specs=pl.BlockSpec((TILE_M, D), lambda i, m: (i, 0)),
        ),
        out_shape=jax.ShapeDtypeStruct((P, D), jnp.float32),
        compiler_params=pltpu.CompilerParams(
            dimension_semantics=("arbitrary",)),
    )(meta1, hs, ws.reshape(P, 1), w1b, b1.reshape(E, 1, FF), w2b,
      b2.reshape(E, 1, D))

    scb = pl.kernel(
        _scb_body,
        out_type=jax.ShapeDtypeStruct((T * K, D), jnp.float32),
        mesh=plsc.VectorSubcoreMesh(core_axis_name="c", subcore_axis_name="s"),
        compiler_params=pltpu.CompilerParams(needs_layout_passes=False),
        scratch_types=[
            pltpu.VMEM((PAIRS,), jnp.int32),
            pltpu.VMEM((PAIRS, D), jnp.float32),
            pltpu.SemaphoreType.DMA,
        ],
    )
    ysg = scb(posf, ys)

    out = pl.pallas_call(
        _k4_body,
        grid=(T // _BT4,),
        in_specs=[
            pl.BlockSpec((_BT4, D), lambda i: (i, 0)),
            pl.BlockSpec((_BT4, K * D), lambda i: (i, 0)),
        ],
        out_specs=pl.BlockSpec((_BT4, D), lambda i: (i, 0)),
        out_shape=jax.ShapeDtypeStruct((T, D), jnp.float32),
    )(x, ysg.reshape(T, K * D))

    return out


# speedup vs baseline: 2.0934x; 2.0934x over previous
import functools

import jax
import jax.numpy as jnp
from jax import lax
from jax.experimental import pallas as pl
from jax.experimental.pallas import tpu as pltpu
from jax.experimental.pallas import tpu_sc as plsc

D = 768
FF = 3072
E = 8
K = 2
T = 2048

TILE_M = 512
TSH = 9
P = T * K + E * TILE_M
NT = P // TILE_M
TF = 1536
NF = FF // TF

_BT1 = 256
_BT4 = 256

NW = 32
TPW = T // NW


def _k1_body(x_ref, wg_ref, g_ref, b_ref, h_ref, ti_ref, tw_ref):
    x = x_ref[...]
    mu = jnp.mean(x, axis=-1, keepdims=True)
    var = jnp.mean((x - mu) ** 2, axis=-1, keepdims=True)
    h = (x - mu) / jnp.sqrt(var + 1e-5) * g_ref[...] + b_ref[...]
    h_ref[...] = h
    logits = jnp.dot(h, wg_ref[...], preferred_element_type=jnp.float32)
    m = jnp.max(logits, axis=-1, keepdims=True)
    ex = jnp.exp(logits - m)
    p = ex / jnp.sum(ex, axis=-1, keepdims=True)
    ei = jax.lax.broadcasted_iota(jnp.int32, (_BT1, E), 1)
    v1 = jnp.max(p, axis=-1, keepdims=True)
    i1 = jnp.min(jnp.where(p == v1, ei, E), axis=-1, keepdims=True)
    p2 = jnp.where(ei == i1, -1.0, p)
    v2 = jnp.max(p2, axis=-1, keepdims=True)
    i2 = jnp.min(jnp.where(p2 == v2, ei, E), axis=-1, keepdims=True)
    den = v1 + v2 + 1e-9
    ti_ref[...] = jnp.concatenate([i1, i2], axis=1)
    tw_ref[...] = jnp.concatenate([v1 / den, v2 / den], axis=1)


def _route_body(ef_ref, pos_ref, meta_ref):
    e = ef_ref[...]
    n = T * K
    se = jax.lax.broadcasted_iota(jnp.int32, (E, n), 0)
    oh = (e == se).astype(jnp.int32)
    c = oh
    s = 1
    while s < n:
        z = jnp.zeros((E, s), jnp.int32)
        c = c + jnp.concatenate([z, c[:, : n - s]], axis=1)
        s *= 2
    rank = jnp.sum(oh * c, axis=0, keepdims=True) - 1
    counts = c[:, n - 1 : n]
    psz = ((counts + (TILE_M - 1)) >> TSH) << TSH
    q = psz
    s = 1
    while s < E:
        z = jnp.zeros((s, 1), jnp.int32)
        q = q + jnp.concatenate([z, q[: E - s, :]], axis=0)
        s *= 2
    off = q - psz
    pos_ref[...] = jnp.sum(oh * off, axis=0, keepdims=True) + rank
    p_used = q[E - 1 : E, :]
    ti = jax.lax.broadcasted_iota(jnp.int32, (1, 64), 1) * TILE_M
    texp = jnp.zeros((1, 64), jnp.int32)
    for ee in range(1, E):
        texp = texp + (ti >= off[ee : ee + 1, :]).astype(jnp.int32)
    nv = p_used >> TSH
    li = jax.lax.broadcasted_iota(jnp.int32, (1, 64), 1)
    meta_ref[...] = jnp.where(li == 63, nv, texp)


def _sca_body(pos_hbm, h_hbm, hs_hbm, p0_v, p1_v, hv, sem):
    wid = lax.axis_index("s") * 2 + lax.axis_index("c")
    t0 = wid * TPW
    pltpu.sync_copy(pos_hbm.at[pl.ds(t0, TPW)], p0_v)
    pltpu.sync_copy(pos_hbm.at[pl.ds(T + t0, TPW)], p1_v)
    pltpu.sync_copy(h_hbm.at[pl.ds(t0, TPW)], hv)
    pltpu.async_copy(hv, hs_hbm.at[p0_v], sem).wait()
    pltpu.async_copy(hv, hs_hbm.at[p1_v], sem).wait()


def _scb_body(pos_hbm, ys_hbm, ysg_hbm, p0_v, p1_v, rows_v, sem):
    wid = lax.axis_index("s") * 2 + lax.axis_index("c")
    t0 = wid * TPW
    pltpu.sync_copy(pos_hbm.at[pl.ds(t0, TPW)], p0_v)
    pltpu.sync_copy(pos_hbm.at[pl.ds(T + t0, TPW)], p1_v)
    pltpu.async_copy(ys_hbm.at[p0_v], rows_v, sem).wait()
    pltpu.sync_copy(rows_v, ysg_hbm.at[pl.ds(t0, TPW)])
    pltpu.async_copy(ys_hbm.at[p1_v], rows_v, sem).wait()
    pltpu.sync_copy(rows_v, ysg_hbm.at[pl.ds(T + t0, TPW)])


def _k3_body(meta_ref, hs_ref, w1_ref, b1_ref, w2_ref, b2_ref, ys_ref):
    i = pl.program_id(0)
    f = pl.program_id(1)
    nv = meta_ref[63]

    @pl.when(i < nv)
    def _ffn():
        hs_bf = hs_ref[...].astype(jnp.bfloat16)
        a = jnp.maximum(
            jnp.dot(hs_bf, w1_ref[0].astype(jnp.bfloat16),
                    preferred_element_type=jnp.float32) + b1_ref[0], 0.0)
        contrib = jnp.dot(a.astype(jnp.bfloat16),
                          w2_ref[0].astype(jnp.bfloat16),
                          preferred_element_type=jnp.float32)

        @pl.when(f == 0)
        def _init():
            ys_ref[...] = b2_ref[0] + contrib

        @pl.when(f > 0)
        def _acc():
            ys_ref[...] = ys_ref[...] + contrib


def _k4_body(x_ref, w_ref, y0_ref, y1_ref, o_ref):
    o_ref[...] = (x_ref[...] + w_ref[:, 0:1] * y0_ref[0]
                  + w_ref[:, 1:2] * y1_ref[0])


def kernel(x, Wg, W1, b1, W2, b2, gamma, beta):
    g2 = gamma.reshape(1, D)
    bt2 = beta.reshape(1, D)

    h, tidx, tw = pl.pallas_call(
        _k1_body,
        grid=(T // _BT1,),
        in_specs=[
            pl.BlockSpec((_BT1, D), lambda i: (i, 0)),
            pl.BlockSpec((D, E), lambda i: (0, 0)),
            pl.BlockSpec((1, D), lambda i: (0, 0)),
            pl.BlockSpec((1, D), lambda i: (0, 0)),
        ],
        out_specs=[
            pl.BlockSpec((_BT1, D), lambda i: (i, 0)),
            pl.BlockSpec((_BT1, K), lambda i: (i, 0)),
            pl.BlockSpec((_BT1, K), lambda i: (i, 0)),
        ],
        out_shape=[
            jax.ShapeDtypeStruct((T, D), jnp.float32),
            jax.ShapeDtypeStruct((T, K), jnp.int32),
            jax.ShapeDtypeStruct((T, K), jnp.float32),
        ],
    )(x, Wg, g2, bt2)

    ef = tidx.T.reshape(1, T * K)
    pos1, meta = pl.pallas_call(
        _route_body,
        in_specs=[pl.BlockSpec((1, T * K), lambda: (0, 0))],
        out_specs=[
            pl.BlockSpec((1, T * K), lambda: (0, 0)),
            pl.BlockSpec((1, 64), lambda: (0, 0)),
        ],
        out_shape=[
            jax.ShapeDtypeStruct((1, T * K), jnp.int32),
            jax.ShapeDtypeStruct((1, 64), jnp.int32),
        ],
    )(ef)

    posf = pos1.reshape(T * K)
    meta1 = meta.reshape(64)

    sca = pl.kernel(
        _sca_body,
        out_type=jax.ShapeDtypeStruct((P, D), jnp.float32),
        mesh=plsc.VectorSubcoreMesh(core_axis_name="c", subcore_axis_name="s"),
        compiler_params=pltpu.CompilerParams(needs_layout_passes=False),
        scratch_types=[
            pltpu.VMEM((TPW,), jnp.int32),
            pltpu.VMEM((TPW,), jnp.int32),
            pltpu.VMEM((TPW, D), jnp.float32),
            pltpu.SemaphoreType.DMA,
        ],
    )
    hs = sca(posf, h)

    ys = pl.pallas_call(
        _k3_body,
        grid_spec=pltpu.PrefetchScalarGridSpec(
            num_scalar_prefetch=1,
            grid=(NT, NF),
            in_specs=[
                pl.BlockSpec((TILE_M, D), lambda i, f, m: (i, 0)),
                pl.BlockSpec((1, D, TF), lambda i, f, m: (m[i], 0, f)),
                pl.BlockSpec((1, 1, TF), lambda i, f, m: (m[i], 0, f)),
                pl.BlockSpec((1, TF, D), lambda i, f, m: (m[i], f, 0)),
                pl.BlockSpec((1, 1, D), lambda i, f, m: (m[i], 0, 0)),
            ],
            out_specs=pl.BlockSpec((TILE_M, D), lambda i, f, m: (i, 0)),
        ),
        out_shape=jax.ShapeDtypeStruct((P, D), jnp.float32),
        compiler_params=pltpu.CompilerParams(
            dimension_semantics=("arbitrary", "arbitrary")),
    )(meta1, hs, W1, b1.reshape(E, 1, FF), W2, b2.reshape(E, 1, D))

    scb = pl.kernel(
        _scb_body,
        out_type=jax.ShapeDtypeStruct((T * K, D), jnp.float32),
        mesh=plsc.VectorSubcoreMesh(core_axis_name="c", subcore_axis_name="s"),
        compiler_params=pltpu.CompilerParams(needs_layout_passes=False),
        scratch_types=[
            pltpu.VMEM((TPW,), jnp.int32),
            pltpu.VMEM((TPW,), jnp.int32),
            pltpu.VMEM((TPW, D), jnp.float32),
            pltpu.SemaphoreType.DMA,
        ],
    )
    ysg = scb(posf, ys)
    ysg3 = ysg.reshape(K, T, D)

    out = pl.pallas_call(
        _k4_body,
        grid=(T // _BT4,),
        in_specs=[
            pl.BlockSpec((_BT4, D), lambda i: (i, 0)),
            pl.BlockSpec((_BT4, K), lambda i: (i, 0)),
            pl.BlockSpec((1, _BT4, D), lambda i: (0, i, 0)),
            pl.BlockSpec((1, _BT4, D), lambda i: (1, i, 0)),
        ],
        out_specs=pl.BlockSpec((_BT4, D), lambda i: (i, 0)),
        out_shape=jax.ShapeDtypeStruct((T, D), jnp.float32),
    )(x, tw, ysg3, ysg3)

    return out
